# baseline (speedup 1.0000x reference)
import jax
import jax.numpy as jnp
from jax import lax
from jax.experimental import pallas as pl
from jax.experimental.pallas import tpu as pltpu
from jax.experimental.pallas import tpu_sc as plsc

F32 = jnp.float32
N_MASK = 8192

F_TILE = 1024
D_TILE = 8192
N_STEPS = N_MASK // F_TILE


def _sc_compact_body(ch1_hbm, flat_hbm, vals, outv):
    cid = lax.axis_index("c")
    sid = lax.axis_index("s")

    @pl.when(jnp.logical_and(cid == 0, sid == 0))
    def _():
        pltpu.sync_copy(ch1_hbm, vals)
        lane = lax.iota(jnp.int32, 16)
        zero16 = jnp.zeros((16,), jnp.int32)
        one16 = jnp.ones((16,), jnp.int32)

        def zb(i, c):
            for u in range(4):
                outv[pl.ds((i * 4 + u) * 16, 16)] = zero16
            return c

        lax.fori_loop(0, N_MASK // 64, zb, 0)

        def comp(i, off):
            for u in range(4):
                k = i * 4 + u
                v = vals[pl.ds(k * 16, 16)]
                m = v == 1.0
                mi = jnp.where(m, one16, zero16)
                pos = off + plsc.cumsum(mi) - 1
                plsc.store_scatter(outv, [pos], lane + k * 16, mask=m)
                off = off + plsc.all_reduce_population_count(m)
            return off

        lax.fori_loop(0, N_MASK // 64, comp, zero16)
        pltpu.sync_copy(outv, flat_hbm)


def _sc_compact(ch1):
    mesh = plsc.VectorSubcoreMesh(core_axis_name="c", subcore_axis_name="s")
    kfn = pl.kernel(
        _sc_compact_body,
        mesh=mesh,
        out_type=jax.ShapeDtypeStruct((N_MASK,), jnp.int32),
        scratch_types=[
            pltpu.VMEM((N_MASK,), F32),
            pltpu.VMEM((N_MASK,), jnp.int32),
        ],
        compiler_params=pltpu.CompilerParams(needs_layout_passes=False),
    )
    return kfn(ch1)


def _fused_body(f_ref, x_ref, ag_ref, w0_ref, b0_ref, w1_ref, b1_ref, w2_ref,
                b2_ref, dw0_ref, db0_ref, dw1_ref, db1_ref,
                front_ref, dist_ref, agent_ref):
    b0 = b0_ref[...]
    b1 = b1_ref[...]
    b2 = b2_ref[...]
    w1 = w1_ref[...]
    w2 = w2_ref[...]

    ff = f_ref[...].astype(F32)
    ri = lax.broadcasted_iota(jnp.int32, (128, 128), 0)
    ci = lax.broadcasted_iota(jnp.int32, (128, 128), 1)
    eye = (ri == ci).astype(F32)
    ones_col = jnp.ones((128, 1), F32)
    gf = (ff[:, None, :] * eye[None, :, :]).reshape(F_TILE, 128)
    fcol = jnp.dot(gf, ones_col, preferred_element_type=F32,
                   precision=lax.Precision.HIGHEST)
    fd32 = jnp.floor(fcol * (1.0 / 32.0))
    bidx = jnp.floor(fcol * (1.0 / 1024.0))
    x = fcol - fd32 * 32.0
    y = fd32 - bidx * 32.0
    h0 = jnp.maximum(x * w0_ref[0:1, :] + y * w0_ref[1:2, :] + b0, 0.0)
    h1 = jnp.maximum(jnp.dot(h0, w1, preferred_element_type=F32) + b1, 0.0)
    rows = jnp.dot(h1, w2, preferred_element_type=F32) + b2
    c0 = jnp.maximum(b0, 0.0)
    c1 = jnp.maximum(jnp.dot(c0, w1, preferred_element_type=F32) + b1, 0.0)
    cc = jnp.dot(c1, w2, preferred_element_type=F32) + b2
    cfull = jnp.broadcast_to(cc, rows.shape)
    for b in range(8):
        front_ref[b, :, :] = jnp.where(bidx == float(b), rows, cfull)

    xd = x_ref[...]
    g = (xd[:, None, :] * eye[None, :, :]).reshape(D_TILE, 128)
    w0rep = jnp.broadcast_to(dw0_ref[...], (128, 256))
    h = jnp.maximum(jnp.dot(g, w0rep, preferred_element_type=F32) + db0_ref[...], 0.0)
    dist_ref[...] = jnp.dot(h, dw1_ref[...], preferred_element_type=F32) + db1_ref[...]

    @pl.when(pl.program_id(0) == 0)
    def _():
        g = ag_ref[...]
        g0 = g[:, 0:1] * w0_ref[0:1, :]
        g0 = g0 + g[:, 1:2] * w0_ref[1:2, :]
        g0 = g0 + g[:, 2:3] * w0_ref[2:3, :]
        g0 = g0 + g[:, 3:4] * w0_ref[3:4, :]
        g0 = jnp.maximum(g0 + b0, 0.0)
        g1 = jnp.maximum(jnp.dot(g0, w1, preferred_element_type=F32) + b1, 0.0)
        agent_ref[...] = jnp.dot(g1, w2, preferred_element_type=F32) + b2


def _fused_tc(flat2d, x2d, agfeats, w0, b0, w1, b1, w2, b2, dw0, db0, dw1, db1):
    n_agent = agfeats.shape[0]
    return pl.pallas_call(
        _fused_body,
        grid=(N_STEPS,),
        in_specs=[
            pl.BlockSpec((F_TILE // 128, 128), lambda i: (i, 0)),
            pl.BlockSpec((D_TILE // 128, 128), lambda i: (i, 0)),
            pl.BlockSpec((n_agent, 4), lambda i: (0, 0)),
            pl.BlockSpec((4, 128), lambda i: (0, 0)),
            pl.BlockSpec((1, 128), lambda i: (0, 0)),
            pl.BlockSpec((128, 256), lambda i: (0, 0)),
            pl.BlockSpec((1, 256), lambda i: (0, 0)),
            pl.BlockSpec((256, 256), lambda i: (0, 0)),
            pl.BlockSpec((1, 256), lambda i: (0, 0)),
            pl.BlockSpec((1, 256), lambda i: (0, 0)),
            pl.BlockSpec((1, 256), lambda i: (0, 0)),
            pl.BlockSpec((256, 256), lambda i: (0, 0)),
            pl.BlockSpec((1, 256), lambda i: (0, 0)),
        ],
        out_specs=[
            pl.BlockSpec((8, F_TILE, 256), lambda i: (0, i, 0)),
            pl.BlockSpec((D_TILE, 256), lambda i: (i, 0)),
            pl.BlockSpec((n_agent, 256), lambda i: (0, 0)),
        ],
        out_shape=[
            jax.ShapeDtypeStruct((8, N_MASK, 256), F32),
            jax.ShapeDtypeStruct((N_STEPS * D_TILE, 256), F32),
            jax.ShapeDtypeStruct((n_agent, 256), F32),
        ],
    )(flat2d, x2d, agfeats, w0, b0, w1, b1, w2, b2, dw0, db0, dw1, db1)


def kernel(inputs, dist, extras, enc_W0, enc_b0, enc_W1, enc_b1, enc_W2, enc_b2,
           dist_W0, dist_b0, dist_W1, dist_b1):
    B = inputs.shape[0]
    A = extras.shape[1]
    L = dist.shape[1]
    ch1 = inputs[:, 1, :, :].reshape(-1)
    flat = _sc_compact(ch1)
    flat2d = flat.reshape(N_MASK // 128, 128)

    b0 = enc_b0.reshape(1, -1)
    b1 = enc_b1.reshape(1, -1)
    b2 = enc_b2.reshape(1, -1)

    frontier_enc, dist_flat, agent_flat = _fused_tc(
        flat2d, dist.reshape(B * L // 128, 128), extras.reshape(B * A, 4),
        enc_W0, b0, enc_W1, b1, enc_W2, b2,
        dist_W0, dist_b0.reshape(1, -1), dist_W1, dist_b1.reshape(1, -1))
    return (frontier_enc, agent_flat.reshape(B, A, 256),
            dist_flat.reshape(B, L, 256))


# speedup vs baseline: 1.0022x; 1.0022x over previous
import jax
import jax.numpy as jnp
from jax import lax
from jax.experimental import pallas as pl
from jax.experimental.pallas import tpu as pltpu
from jax.experimental.pallas import tpu_sc as plsc

F32 = jnp.float32
N_MASK = 8192

F_TILE = 1024
D_TILE = 8192
N_STEPS = N_MASK // F_TILE


def _sc_compact_body(ch1_hbm, flat_hbm, vals, outv):
    cid = lax.axis_index("c")
    sid = lax.axis_index("s")

    @pl.when(jnp.logical_and(cid == 0, sid == 0))
    def _():
        pltpu.sync_copy(ch1_hbm, vals)
        lane = lax.iota(jnp.int32, 16)
        zero16 = jnp.zeros((16,), jnp.int32)
        one16 = jnp.ones((16,), jnp.int32)

        def zb(i, c):
            for u in range(4):
                outv[pl.ds((i * 4 + u) * 16, 16)] = zero16
            return c

        lax.fori_loop(0, N_MASK // 64, zb, 0)

        def comp(i, off):
            for u in range(4):
                k = i * 4 + u
                v = vals[pl.ds(k * 16, 16)]
                m = v == 1.0
                mi = jnp.where(m, one16, zero16)
                pos = off + plsc.cumsum(mi) - 1
                plsc.store_scatter(outv, [pos], lane + k * 16, mask=m)
                off = off + plsc.all_reduce_population_count(m)
            return off

        lax.fori_loop(0, N_MASK // 64, comp, zero16)
        pltpu.sync_copy(outv, flat_hbm)


def _sc_compact(ch1):
    mesh = plsc.VectorSubcoreMesh(core_axis_name="c", subcore_axis_name="s")
    kfn = pl.kernel(
        _sc_compact_body,
        mesh=mesh,
        out_type=jax.ShapeDtypeStruct((N_MASK,), jnp.int32),
        scratch_types=[
            pltpu.VMEM((N_MASK,), F32),
            pltpu.VMEM((N_MASK,), jnp.int32),
        ],
        compiler_params=pltpu.CompilerParams(needs_layout_passes=False),
    )
    return kfn(ch1)


def _fused_body(f_ref, x_ref, ag_ref, w0_ref, b0_ref, w1_ref, b1_ref, w2_ref,
                b2_ref, dw0_ref, db0_ref, dw1_ref, db1_ref,
                front_ref, dist_ref, agent_ref):
    b0 = b0_ref[...]
    b1 = b1_ref[...]
    b2 = b2_ref[...]
    w1 = w1_ref[...]
    w2 = w2_ref[...]

    ff = f_ref[...].astype(F32)
    ri = lax.broadcasted_iota(jnp.int32, (128, 128), 0)
    ci = lax.broadcasted_iota(jnp.int32, (128, 128), 1)
    eye = (ri == ci).astype(F32)
    ones_col = jnp.ones((128, 1), F32)
    gf = (ff[:, None, :] * eye[None, :, :]).reshape(F_TILE, 128)
    fcol = jnp.dot(gf, ones_col, preferred_element_type=F32,
                   precision=lax.Precision.HIGHEST)
    fd32 = jnp.floor(fcol * (1.0 / 32.0))
    bidx = jnp.floor(fcol * (1.0 / 1024.0))
    x = fcol - fd32 * 32.0
    y = fd32 - bidx * 32.0
    h0 = jnp.maximum(x * w0_ref[0:1, :] + y * w0_ref[1:2, :] + b0, 0.0)
    h1 = jnp.maximum(jnp.dot(h0, w1, preferred_element_type=F32) + b1, 0.0)
    rows = jnp.dot(h1, w2, preferred_element_type=F32) + b2
    c0 = jnp.maximum(b0, 0.0)
    c1 = jnp.maximum(jnp.dot(c0, w1, preferred_element_type=F32) + b1, 0.0)
    cc = jnp.dot(c1, w2, preferred_element_type=F32) + b2
    cfull = jnp.broadcast_to(cc, rows.shape)
    for b in range(8):
        front_ref[b, :, :] = jnp.where(bidx == float(b), rows, cfull)

    xd = x_ref[...]
    g = (xd[:, None, :] * eye[None, :, :]).reshape(D_TILE, 128)
    w0rep = jnp.broadcast_to(dw0_ref[...], (128, 256))
    h = jnp.maximum(jnp.dot(g, w0rep, preferred_element_type=F32) + db0_ref[...], 0.0)
    dist_ref[...] = jnp.dot(h, dw1_ref[...], preferred_element_type=F32) + db1_ref[...]

    @pl.when(pl.program_id(0) == 0)
    def _():
        g = ag_ref[...]
        g0 = g[:, 0:1] * w0_ref[0:1, :]
        g0 = g0 + g[:, 1:2] * w0_ref[1:2, :]
        g0 = g0 + g[:, 2:3] * w0_ref[2:3, :]
        g0 = g0 + g[:, 3:4] * w0_ref[3:4, :]
        g0 = jnp.maximum(g0 + b0, 0.0)
        g1 = jnp.maximum(jnp.dot(g0, w1, preferred_element_type=F32) + b1, 0.0)
        agent_ref[...] = jnp.dot(g1, w2, preferred_element_type=F32) + b2


def _fused_tc(flat2d, x2d, agfeats, w0, b0, w1, b1, w2, b2, dw0, db0, dw1, db1):
    n_agent = agfeats.shape[0]
    return pl.pallas_call(
        _fused_body,
        grid=(N_STEPS,),
        in_specs=[
            pl.BlockSpec((F_TILE // 128, 128), lambda i: (i, 0)),
            pl.BlockSpec((D_TILE // 128, 128), lambda i: (i, 0)),
            pl.BlockSpec((n_agent, 4), lambda i: (0, 0)),
            pl.BlockSpec((4, 128), lambda i: (0, 0)),
            pl.BlockSpec((1, 128), lambda i: (0, 0)),
            pl.BlockSpec((128, 256), lambda i: (0, 0)),
            pl.BlockSpec((1, 256), lambda i: (0, 0)),
            pl.BlockSpec((256, 256), lambda i: (0, 0)),
            pl.BlockSpec((1, 256), lambda i: (0, 0)),
            pl.BlockSpec((1, 256), lambda i: (0, 0)),
            pl.BlockSpec((1, 256), lambda i: (0, 0)),
            pl.BlockSpec((256, 256), lambda i: (0, 0)),
            pl.BlockSpec((1, 256), lambda i: (0, 0)),
        ],
        out_specs=[
            pl.BlockSpec((8, F_TILE, 256), lambda i: (0, i, 0)),
            pl.BlockSpec((D_TILE, 256), lambda i: (i, 0)),
            pl.BlockSpec((n_agent, 256), lambda i: (0, 0)),
        ],
        out_shape=[
            jax.ShapeDtypeStruct((8, N_MASK, 256), F32),
            jax.ShapeDtypeStruct((N_STEPS * D_TILE, 256), F32),
            jax.ShapeDtypeStruct((n_agent, 256), F32),
        ],
        compiler_params=pltpu.CompilerParams(allow_input_fusion=[True, True] + [False] * 11),
    )(flat2d, x2d, agfeats, w0, b0, w1, b1, w2, b2, dw0, db0, dw1, db1)


def kernel(inputs, dist, extras, enc_W0, enc_b0, enc_W1, enc_b1, enc_W2, enc_b2,
           dist_W0, dist_b0, dist_W1, dist_b1):
    B = inputs.shape[0]
    A = extras.shape[1]
    L = dist.shape[1]
    ch1 = inputs[:, 1, :, :].reshape(-1)
    flat = _sc_compact(ch1)
    flat2d = flat.reshape(N_MASK // 128, 128)

    b0 = enc_b0.reshape(1, -1)
    b1 = enc_b1.reshape(1, -1)
    b2 = enc_b2.reshape(1, -1)

    frontier_enc, dist_flat, agent_flat = _fused_tc(
        flat2d, dist.reshape(B * L // 128, 128), extras.reshape(B * A, 4),
        enc_W0, b0, enc_W1, b1, enc_W2, b2,
        dist_W0, dist_b0.reshape(1, -1), dist_W1, dist_b1.reshape(1, -1))
    return (frontier_enc, agent_flat.reshape(B, A, 256),
            dist_flat.reshape(B, L, 256))
